# Initial kernel scaffold; baseline (speedup 1.0000x reference)
import jax
import jax.numpy as jnp
from jax.experimental import pallas as pl


def kernel(x, W_router, W1, W2):
    raise NotImplementedError("write your pallas kernel here")


# speedup vs baseline: 2.0164x; 2.0164x over previous
import functools

import jax
import jax.numpy as jnp
from jax.experimental import pallas as pl
from jax.experimental.pallas import tpu as pltpu

D_MODEL = 768
D_FF = 4 * D_MODEL
NUM_EXPERTS = 8
TOP_K = 2
N_TOKENS = 2048


def _router_body(x_ref, wr_ref, wtok_ref):
    x = x_ref[...]
    logits = jax.lax.dot_general(
        x, wr_ref[...], (((1,), (1,)), ((), ())),
        preferred_element_type=jnp.float32)
    m = jnp.max(logits, axis=-1, keepdims=True)
    p = jnp.exp(logits - m)
    p = p / jnp.sum(p, axis=-1, keepdims=True)
    lane = jax.lax.broadcasted_iota(jnp.int32, p.shape, 1)
    t1 = jnp.max(p, axis=-1, keepdims=True)
    a1 = jnp.argmax(p, axis=-1)[:, None]
    p2 = jnp.where(lane == a1, -jnp.inf, p)
    t2 = jnp.max(p2, axis=-1, keepdims=True)
    a2 = jnp.argmax(p2, axis=-1)[:, None]
    s = t1 + t2
    w1 = t1 / s
    w2 = t2 / s
    wtok_ref[...] = jnp.where(lane == a1, w1, 0.0) + jnp.where(lane == a2, w2, 0.0)


def _ffn_body(x_ref, w1_ref, w2_ref, wtok_ref, out_ref):
    e = pl.program_id(1)
    x = x_ref[...]
    h = jax.lax.dot_general(
        x, w1_ref[0], (((1,), (1,)), ((), ())),
        preferred_element_type=jnp.float32)
    h = 0.5 * h * (1.0 + jax.lax.erf(h * 0.7071067811865476))
    y = jax.lax.dot_general(
        h, w2_ref[0], (((1,), (1,)), ((), ())),
        preferred_element_type=jnp.float32)
    wtok = wtok_ref[...]
    lane = jax.lax.broadcasted_iota(jnp.int32, wtok.shape, 1)
    wcol = jnp.sum(jnp.where(lane == e, wtok, 0.0), axis=1, keepdims=True)
    y = y * wcol

    @pl.when(e == 0)
    def _():
        out_ref[...] = y

    @pl.when(e != 0)
    def _():
        out_ref[...] += y


def kernel(x, W_router, W1, W2):
    Bm, Tm, C = x.shape
    n = Bm * Tm
    x_flat = x.reshape(n, C)

    wtok = pl.pallas_call(
        _router_body,
        out_shape=jax.ShapeDtypeStruct((n, NUM_EXPERTS), jnp.float32),
    )(x_flat, W_router)

    TM = 256
    grid = (n // TM, NUM_EXPERTS)
    out = pl.pallas_call(
        _ffn_body,
        grid=grid,
        in_specs=[
            pl.BlockSpec((TM, C), lambda t, e: (t, 0)),
            pl.BlockSpec((1, D_FF, C), lambda t, e: (e, 0, 0)),
            pl.BlockSpec((1, C, D_FF), lambda t, e: (e, 0, 0)),
            pl.BlockSpec((TM, NUM_EXPERTS), lambda t, e: (t, 0)),
        ],
        out_specs=pl.BlockSpec((TM, C), lambda t, e: (t, 0)),
        out_shape=jax.ShapeDtypeStruct((n, C), jnp.float32),
        compiler_params=pltpu.CompilerParams(
            dimension_semantics=("arbitrary", "arbitrary"),
        ),
    )(x_flat, W1, W2, wtok)
    return out.reshape(Bm, Tm, C)


# speedup vs baseline: 4.5630x; 2.2630x over previous
import functools

import jax
import jax.numpy as jnp
from jax import lax
from jax.experimental import pallas as pl
from jax.experimental.pallas import tpu as pltpu
from jax.experimental.pallas import tpu_sc as plsc

D = 768
F = 3072
E = 8
N = 2048
NE = 2 * N
TILE = 256
MAX_TILES = NE // TILE + E - 1
PAD_N = MAX_TILES * TILE

_INV_SQRT2 = 0.7071067811865476


def _router_body(x_ref, wr_ref, wts_ref, dest_ref, te_ref, lt_ref):
    x = x_ref[...]
    lg = lax.dot_general(wr_ref[...], x, (((1,), (1,)), ((), ())),
                         preferred_element_type=jnp.float32)
    m = jnp.max(lg, axis=0, keepdims=True)
    p = jnp.exp(lg - m)
    p = p / jnp.sum(p, axis=0, keepdims=True)
    row = lax.broadcasted_iota(jnp.int32, p.shape, 0)
    m1 = jnp.max(p, axis=0, keepdims=True)
    a1 = jnp.min(jnp.where(p == m1, row, E), axis=0, keepdims=True)
    p2 = jnp.where(row == a1, -jnp.inf, p)
    m2 = jnp.max(p2, axis=0, keepdims=True)
    a2 = jnp.min(jnp.where(p2 == m2, row, E), axis=0, keepdims=True)
    s = m1 + m2
    wts_ref[...] = jnp.concatenate([m1 / s, m2 / s], axis=0)

    en = jnp.concatenate([a1, a2], axis=0)
    mm = jnp.concatenate(
        [(en == e).astype(jnp.float32) for e in range(E)], axis=0)
    ra = lax.broadcasted_iota(jnp.int32, (N, N), 0)
    rb = lax.broadcasted_iota(jnp.int32, (N, N), 1)
    tri = jnp.where(ra < rb, 1.0, 0.0)
    pp = lax.dot_general(mm, tri, (((1,), (0,)), ((), ())),
                         preferred_element_type=jnp.float32)

    tot0, cnt, ranks = [], [], []
    for e in range(E):
        t0 = (pp[2 * e : 2 * e + 1, N - 1 : N]
              + mm[2 * e : 2 * e + 1, N - 1 : N])
        t1 = (pp[2 * e + 1 : 2 * e + 2, N - 1 : N]
              + mm[2 * e + 1 : 2 * e + 2, N - 1 : N])
        tot0.append(t0)
        cnt.append(t0 + t1)
        r0 = pp[2 * e : 2 * e + 1, :]
        r1 = pp[2 * e + 1 : 2 * e + 2, :] + t0
        ranks.append(jnp.concatenate([r0, r1], axis=0))

    nt = [jnp.floor((c + (TILE - 1)) * (1.0 / TILE)) for c in cnt]
    ts_incl = []
    acc = nt[0]
    ts_incl.append(acc)
    for e in range(1, E):
        acc = acc + nt[e]
        ts_incl.append(acc)
    live = ts_incl[E - 1]

    dest = jnp.zeros((2, N), jnp.float32)
    for e in range(E):
        start_e = (ts_incl[e] - nt[e]) * float(TILE)
        dest = dest + mm[2 * e : 2 * e + 2, :] * (ranks[e] + start_e)
    dest_ref[...] = dest.astype(jnp.int32)

    mlane = lax.broadcasted_iota(jnp.int32, (1, 32), 1).astype(jnp.float32)
    mclamp = jnp.minimum(mlane, live - 1.0)
    te = jnp.zeros((1, 32), jnp.float32)
    for e in range(E):
        te = te + jnp.where(mclamp >= ts_incl[e], 1.0, 0.0)
    te_ref[...] = te.astype(jnp.int32)
    l16 = lax.broadcasted_iota(jnp.int32, (1, 16), 1)
    lt_ref[...] = jnp.where(l16 == 0, live.astype(jnp.int32), 0)


def _dispatch_body(destf, wflat, x, xg, wbuf,
                   d16, w16v, w16m, row_buf, semA, semB):
    wid = lax.axis_index("s") * 2 + lax.axis_index("c")
    lane = lax.iota(jnp.int32, 16)
    base = wid * 128
    tok0 = lax.rem(wid, 16) * 128
    for c in range(8):
        pltpu.sync_copy(destf.at[pl.ds(base + c * 16, 16)], d16)
        pltpu.sync_copy(wflat.at[pl.ds(base + c * 16, 16)], w16v)
        plsc.store_scatter(w16m, [lane, lane * 0], w16v[...])
        pltpu.sync_copy(x.at[pl.ds(tok0 + c * 16, 16)], row_buf)
        cpA = pltpu.async_copy(row_buf, xg.at[d16], semA)
        cpB = pltpu.async_copy(w16m, wbuf.at[d16], semB)
        cpA.wait()
        cpB.wait()


def _ffn_body(te_ref, lt_ref, xg_ref, w1_ref, w2_ref, wb_ref, y_ref):
    m = pl.program_id(0)

    @pl.when(m < lt_ref[0])
    def _():
        xt = xg_ref[...]
        h = lax.dot_general(xt, w1_ref[0], (((1,), (1,)), ((), ())),
                            preferred_element_type=jnp.float32)
        h = 0.5 * h * (1.0 + lax.erf(h * _INV_SQRT2))
        y = lax.dot_general(h, w2_ref[0], (((1,), (1,)), ((), ())),
                            preferred_element_type=jnp.float32)
        y_ref[...] = y * wb_ref[:, 0:1]


def _combine_body(y, dest0, dest1, out,
                  d0b, d1b, rowsA, rowsB, out_buf, semA, semB):
    wid = lax.axis_index("s") * 2 + lax.axis_index("c")
    tok0 = wid * 64

    def chunk(c, _):
        t0 = tok0 + c * 16
        pltpu.sync_copy(dest0.at[pl.ds(t0, 16)], d0b)
        pltpu.sync_copy(dest1.at[pl.ds(t0, 16)], d1b)
        cpA = pltpu.async_copy(y.at[d0b], rowsA, semA)
        cpB = pltpu.async_copy(y.at[d1b], rowsB, semB)
        cpA.wait()
        cpB.wait()

        def tok(j, _2):
            for l in range(D // 16):
                sl = pl.ds(l * 16, 16)
                out_buf[j, sl] = rowsA[j, sl] + rowsB[j, sl]
            return 0

        lax.fori_loop(0, 16, tok, 0)
        pltpu.sync_copy(out_buf, out.at[pl.ds(t0, 16)])
        return 0

    lax.fori_loop(0, 4, chunk, 0)


def _router_call(x_flat, W_router):
    return pl.pallas_call(
        _router_body,
        out_shape=(jax.ShapeDtypeStruct((2, N), jnp.float32),
                   jax.ShapeDtypeStruct((2, N), jnp.int32),
                   jax.ShapeDtypeStruct((1, 32), jnp.int32),
                   jax.ShapeDtypeStruct((1, 16), jnp.int32)),
    )(x_flat, W_router)


def _ffn_call(te, lt, xg, W1, W2, wbuf):
    grid_spec = pltpu.PrefetchScalarGridSpec(
        num_scalar_prefetch=2,
        grid=(MAX_TILES,),
        in_specs=[
            pl.BlockSpec((TILE, D),
                         lambda m, te, lt: (jnp.minimum(m, lt[0] - 1), 0)),
            pl.BlockSpec((1, F, D),
                         lambda m, te, lt: (te[jnp.minimum(m, lt[0] - 1)], 0, 0)),
            pl.BlockSpec((1, D, F),
                         lambda m, te, lt: (te[jnp.minimum(m, lt[0] - 1)], 0, 0)),
            pl.BlockSpec((TILE, 128),
                         lambda m, te, lt: (jnp.minimum(m, lt[0] - 1), 0)),
        ],
        out_specs=pl.BlockSpec((TILE, D),
                               lambda m, te, lt: (jnp.minimum(m, lt[0] - 1), 0)),
    )
    return pl.pallas_call(
        _ffn_body,
        grid_spec=grid_spec,
        out_shape=jax.ShapeDtypeStruct((PAD_N, D), jnp.float32),
        compiler_params=pltpu.CompilerParams(
            dimension_semantics=("arbitrary",)),
    )(te, lt, xg, W1, W2, wbuf)


@jax.jit
def _moe(x_flat, W_router, W1, W2):
    wts, dest, te, lt = _router_call(x_flat, W_router)

    dispatch = pl.kernel(
        _dispatch_body,
        out_type=(jax.ShapeDtypeStruct((PAD_N, D), jnp.float32),
                  jax.ShapeDtypeStruct((PAD_N, 128), jnp.float32)),
        mesh=plsc.VectorSubcoreMesh(core_axis_name="c", subcore_axis_name="s",
                                    num_cores=2, num_subcores=16),
        scratch_types=[
            pltpu.VMEM((16,), jnp.int32),
            pltpu.VMEM((16,), jnp.float32),
            pltpu.VMEM((16, 128), jnp.float32),
            pltpu.VMEM((16, D), jnp.float32),
            pltpu.SemaphoreType.DMA,
            pltpu.SemaphoreType.DMA,
        ],
        compiler_params=pltpu.CompilerParams(needs_layout_passes=False),
    )
    xg, wbuf = dispatch(dest.reshape(NE), wts.reshape(NE), x_flat)

    y_buf = _ffn_call(te.reshape(32), lt.reshape(16), xg, W1, W2, wbuf)

    combine = pl.kernel(
        _combine_body,
        out_type=jax.ShapeDtypeStruct((N, D), jnp.float32),
        mesh=plsc.VectorSubcoreMesh(core_axis_name="c", subcore_axis_name="s",
                                    num_cores=2, num_subcores=16),
        scratch_types=[
            pltpu.VMEM((16,), jnp.int32),
            pltpu.VMEM((16,), jnp.int32),
            pltpu.VMEM((16, D), jnp.float32),
            pltpu.VMEM((16, D), jnp.float32),
            pltpu.VMEM((16, D), jnp.float32),
            pltpu.SemaphoreType.DMA,
            pltpu.SemaphoreType.DMA,
        ],
        compiler_params=pltpu.CompilerParams(needs_layout_passes=False),
    )
    out = combine(y_buf, dest.reshape(NE)[:N], dest.reshape(NE)[N:])
    return out


def kernel(x, W_router, W1, W2):
    Bm, Tm, C = x.shape
    x_flat = x.reshape(Bm * Tm, C)
    out = _moe(x_flat, W_router, W1, W2)
    return out.reshape(Bm, Tm, C)
